# Initial kernel scaffold; baseline (speedup 1.0000x reference)
import jax
import jax.numpy as jnp
from jax.experimental import pallas as pl


def kernel(x, edge_index, edge_weight, W1, b1, W2, b2):
    raise NotImplementedError("write your pallas kernel here")


# speedup vs baseline: 26.2028x; 26.2028x over previous
import functools

import jax
import jax.numpy as jnp
from jax import lax
from jax.experimental import pallas as pl
from jax.experimental.pallas import tpu as pltpu
from jax.experimental.pallas import tpu_sc as plsc

_NC = 2
_NS = 16
_NW = _NC * _NS
_L = 16


def _edge_scatter_sc(n, ch, k, with_table):
    rows_per_tile = n // _NS
    mesh = plsc.VectorSubcoreMesh(core_axis_name="c", subcore_axis_name="s")

    scratch = [
        pltpu.VMEM((ch, k), jnp.int32),
        pltpu.VMEM((ch, k), jnp.float32),
        pltpu.VMEM((k, _L), jnp.float32),
        pltpu.VMEM((rows_per_tile, _L), jnp.float32),
        pltpu.VMEM_SHARED((n, _L), jnp.float32),
        pltpu.SemaphoreType.DMA,
    ]
    if with_table:
        scratch.insert(0, pltpu.VMEM((ch, k), jnp.int32))

    def body(*refs):
        if with_table:
            (tab_hbm, row_hbm, col_hbm, ew_hbm, out_hbm,
             row_v, col_v, ew_v, msg_v, buf_v, acc, sem) = refs
        else:
            (col_hbm, ew_hbm, out_hbm,
             col_v, ew_v, msg_v, buf_v, acc, sem) = refs
        cid = lax.axis_index("c")
        sid = lax.axis_index("s")
        wid = sid * _NC + cid
        base = sid * rows_per_tile

        @pl.loop(0, rows_per_tile)
        def _zero(i):
            buf_v[i] = jnp.zeros((_L,), jnp.float32)

        pltpu.sync_copy(buf_v, acc.at[pl.ds(base, rows_per_tile)])

        if with_table:
            pltpu.sync_copy(row_hbm.at[wid], row_v)
        pltpu.sync_copy(col_hbm.at[wid], col_v)
        pltpu.sync_copy(ew_hbm.at[wid], ew_v)
        plsc.subcore_barrier()

        @pl.loop(0, ch)
        def _chunk(j):
            if with_table:
                pltpu.async_copy(tab_hbm.at[row_v.at[j]], msg_v, sem).wait()
                for g in range(k // _L):
                    ewv = ew_v[j, pl.ds(g * _L, _L)]
                    for t in range(_L):
                        i = g * _L + t
                        msg_v[i] = msg_v[i] * ewv[t]
            else:
                for g in range(k // _L):
                    ewv = ew_v[j, pl.ds(g * _L, _L)]
                    for t in range(_L):
                        msg_v[g * _L + t] = jnp.broadcast_to(ewv[t], (_L,))
            pltpu.sync_copy(msg_v, acc.at[col_v.at[j]], add=True)

        plsc.subcore_barrier()
        pltpu.sync_copy(acc.at[pl.ds(base, rows_per_tile)], buf_v)
        pltpu.sync_copy(buf_v, out_hbm.at[cid, pl.ds(base, rows_per_tile)])

    return pl.kernel(
        body,
        out_type=jax.ShapeDtypeStruct((_NC, n, _L), jnp.float32),
        mesh=mesh,
        scratch_types=scratch,
        compiler_params=pltpu.CompilerParams(use_tc_tiling_on_sc=False),
    )


def _tc1_body(x_ref, w1_ref, degp_ref, h1p_ref, dinv_ref):
    deg = degp_ref[0] + degp_ref[1] + 1.0
    dinv = jnp.where(deg > 0, lax.rsqrt(deg), 0.0)
    h1 = jnp.dot(x_ref[...], w1_ref[...], preferred_element_type=jnp.float32)
    h1p_ref[...] = dinv * h1
    dinv_ref[...] = dinv


def _tc2_body(s1p_ref, dinv_ref, h1p_ref, b1_ref, w2_ref, h2p_ref):
    dinv = dinv_ref[...]
    pre = dinv * (s1p_ref[0] + s1p_ref[1] + h1p_ref[...]) + b1_ref[...]
    out1 = jnp.maximum(pre, 0.0)
    h2 = jnp.dot(out1, w2_ref[...], preferred_element_type=jnp.float32)
    h2p_ref[...] = dinv * h2


def _tc3_body(s2p_ref, dinv_ref, h2p_ref, b2_ref, y_ref):
    pre = dinv_ref[...] * (s2p_ref[0] + s2p_ref[1] + h2p_ref[...]) + b2_ref[...]
    m = jnp.max(pre, axis=1, keepdims=True)
    shifted = pre - m
    lse = jnp.log(jnp.sum(jnp.exp(shifted), axis=1, keepdims=True))
    y_ref[...] = shifted - lse


def _tc_call(body, out_shapes):
    return pl.pallas_call(body, out_shape=out_shapes)


@functools.partial(jax.jit, static_argnames=())
def kernel(x, edge_index, edge_weight, W1, b1, W2, b2):
    n = x.shape[0]
    e = edge_weight.shape[0]
    k = 80
    assert e % (_NW * k) == 0 and n % _NS == 0
    ch = e // (_NW * k)

    row3 = edge_index[0].reshape(_NW, ch, k)
    col3 = edge_index[1].reshape(_NW, ch, k)
    ew3 = edge_weight.reshape(_NW, ch, k)
    b1r = b1.reshape(1, _L)
    b2r = b2.reshape(1, _L)

    deg_sc = _edge_scatter_sc(n, ch, k, with_table=False)
    msg_sc = _edge_scatter_sc(n, ch, k, with_table=True)
    f16 = jax.ShapeDtypeStruct((n, _L), jnp.float32)

    degp = deg_sc(col3, ew3)
    h1p, dinv = _tc_call(_tc1_body, (f16, f16))(x, W1, degp)
    s1p = msg_sc(h1p, row3, col3, ew3)
    h2p = _tc_call(_tc2_body, f16)(s1p, dinv, h1p, b1r, W2)
    s2p = msg_sc(h2p, row3, col3, ew3)
    return _tc_call(_tc3_body, f16)(s2p, dinv, h2p, b2r)


# speedup vs baseline: 37.3666x; 1.4261x over previous
import functools

import jax
import jax.numpy as jnp
from jax import lax
from jax.experimental import pallas as pl
from jax.experimental.pallas import tpu as pltpu
from jax.experimental.pallas import tpu_sc as plsc

_NC = 2
_NS = 16
_NW = _NC * _NS
_L = 16


def _edge_scatter_sc(n, ch, k, with_table):
    rows_per_tile = n // _NS
    mesh = plsc.VectorSubcoreMesh(core_axis_name="c", subcore_axis_name="s")

    assert ch % 2 == 1, "pipelined pair loop assumes an odd chunk count"

    scratch = [
        pltpu.VMEM((ch, k), jnp.int32),
        pltpu.VMEM((ch, k), jnp.float32),
        pltpu.VMEM((k, _L), jnp.float32),
        pltpu.VMEM((k, _L), jnp.float32),
        pltpu.VMEM((rows_per_tile, _L), jnp.float32),
        pltpu.VMEM_SHARED((n, _L), jnp.float32),
        pltpu.SemaphoreType.DMA,
        pltpu.SemaphoreType.DMA,
        pltpu.SemaphoreType.DMA,
        pltpu.SemaphoreType.DMA,
    ]
    if with_table:
        scratch.insert(0, pltpu.VMEM((ch, k), jnp.int32))

    def body(*refs):
        if with_table:
            (tab_hbm, row_hbm, col_hbm, ew_hbm, out_hbm,
             row_v, col_v, ew_v, msg_a, msg_b, buf_v, acc,
             gsem_a, gsem_b, ssem_a, ssem_b) = refs
        else:
            (col_hbm, ew_hbm, out_hbm,
             col_v, ew_v, msg_a, msg_b, buf_v, acc,
             gsem_a, gsem_b, ssem_a, ssem_b) = refs
        cid = lax.axis_index("c")
        sid = lax.axis_index("s")
        wid = sid * _NC + cid
        base = sid * rows_per_tile

        @pl.loop(0, rows_per_tile)
        def _zero(i):
            buf_v[i] = jnp.zeros((_L,), jnp.float32)

        pltpu.sync_copy(buf_v, acc.at[pl.ds(base, rows_per_tile)])

        if with_table:
            pltpu.sync_copy(row_hbm.at[wid], row_v)
        pltpu.sync_copy(col_hbm.at[wid], col_v)
        pltpu.sync_copy(ew_hbm.at[wid], ew_v)
        plsc.subcore_barrier()

        def gather(j, buf, sem):
            if with_table:
                pltpu.async_copy(tab_hbm.at[row_v.at[j]], buf, sem)

        def gather_wait(j, buf, sem):
            if with_table:
                pltpu.make_async_copy(tab_hbm.at[row_v.at[j]], buf, sem).wait()

        def scale(j, buf):
            for g in range(k // _L):
                ewv = ew_v[j, pl.ds(g * _L, _L)]
                for t in range(_L):
                    i = g * _L + t
                    if with_table:
                        buf[i] = buf[i] * ewv[t]
                    else:
                        buf[i] = jnp.broadcast_to(ewv[t], (_L,))

        def scatter(j, buf, sem):
            pltpu.async_copy(buf, acc.at[col_v.at[j]], sem, add=True)

        def scatter_wait(j, buf, sem):
            pltpu.make_async_copy(buf, acc.at[col_v.at[j]], sem).wait()

        gather(0, msg_a, gsem_a)

        @pl.loop(0, (ch - 1) // 2)
        def _pair(i):
            j0 = 2 * i
            j1 = j0 + 1
            jn = j0 + 2
            gather(j1, msg_b, gsem_b)
            gather_wait(j0, msg_a, gsem_a)
            scale(j0, msg_a)
            scatter(j0, msg_a, ssem_a)
            gather_wait(j1, msg_b, gsem_b)
            scale(j1, msg_b)
            scatter(j1, msg_b, ssem_b)
            scatter_wait(j0, msg_a, ssem_a)
            gather(jn, msg_a, gsem_a)
            scatter_wait(j1, msg_b, ssem_b)

        gather_wait(ch - 1, msg_a, gsem_a)
        scale(ch - 1, msg_a)
        scatter(ch - 1, msg_a, ssem_a)
        scatter_wait(ch - 1, msg_a, ssem_a)

        plsc.subcore_barrier()
        pltpu.sync_copy(acc.at[pl.ds(base, rows_per_tile)], buf_v)
        pltpu.sync_copy(buf_v, out_hbm.at[cid, pl.ds(base, rows_per_tile)])

    return pl.kernel(
        body,
        out_type=jax.ShapeDtypeStruct((_NC, n, _L), jnp.float32),
        mesh=mesh,
        scratch_types=scratch,
        compiler_params=pltpu.CompilerParams(use_tc_tiling_on_sc=False),
    )


def _tc1_body(x_ref, w1_ref, degp_ref, h1p_ref, dinv_ref):
    deg = degp_ref[0] + degp_ref[1] + 1.0
    dinv = jnp.where(deg > 0, lax.rsqrt(deg), 0.0)
    h1 = jnp.dot(x_ref[...], w1_ref[...], preferred_element_type=jnp.float32)
    h1p_ref[...] = dinv * h1
    dinv_ref[...] = dinv


def _tc2_body(s1p_ref, dinv_ref, h1p_ref, b1_ref, w2_ref, h2p_ref):
    dinv = dinv_ref[...]
    pre = dinv * (s1p_ref[0] + s1p_ref[1] + h1p_ref[...]) + b1_ref[...]
    out1 = jnp.maximum(pre, 0.0)
    h2 = jnp.dot(out1, w2_ref[...], preferred_element_type=jnp.float32)
    h2p_ref[...] = dinv * h2


def _tc3_body(s2p_ref, dinv_ref, h2p_ref, b2_ref, y_ref):
    pre = dinv_ref[...] * (s2p_ref[0] + s2p_ref[1] + h2p_ref[...]) + b2_ref[...]
    m = jnp.max(pre, axis=1, keepdims=True)
    shifted = pre - m
    lse = jnp.log(jnp.sum(jnp.exp(shifted), axis=1, keepdims=True))
    y_ref[...] = shifted - lse


def _tc_call(body, out_shapes):
    return pl.pallas_call(body, out_shape=out_shapes)


@functools.partial(jax.jit, static_argnames=())
def kernel(x, edge_index, edge_weight, W1, b1, W2, b2):
    n = x.shape[0]
    e = edge_weight.shape[0]
    k = 80
    assert e % (_NW * k) == 0 and n % _NS == 0
    ch = e // (_NW * k)

    row3 = edge_index[0].reshape(_NW, ch, k)
    col3 = edge_index[1].reshape(_NW, ch, k)
    ew3 = edge_weight.reshape(_NW, ch, k)
    b1r = b1.reshape(1, _L)
    b2r = b2.reshape(1, _L)

    deg_sc = _edge_scatter_sc(n, ch, k, with_table=False)
    msg_sc = _edge_scatter_sc(n, ch, k, with_table=True)
    f16 = jax.ShapeDtypeStruct((n, _L), jnp.float32)

    degp = deg_sc(col3, ew3)
    h1p, dinv = _tc_call(_tc1_body, (f16, f16))(x, W1, degp)
    s1p = msg_sc(h1p, row3, col3, ew3)
    h2p = _tc_call(_tc2_body, f16)(s1p, dinv, h1p, b1r, W2)
    s2p = msg_sc(h2p, row3, col3, ew3)
    return _tc_call(_tc3_body, f16)(s2p, dinv, h2p, b2r)


# speedup vs baseline: 51.6926x; 1.3834x over previous
import functools

import jax
import jax.numpy as jnp
from jax import lax
from jax.experimental import pallas as pl
from jax.experimental.pallas import tpu as pltpu
from jax.experimental.pallas import tpu_sc as plsc

_NC = 2
_NS = 16
_NW = _NC * _NS
_L = 16


def _edge_scatter_sc(n, ch, k, with_table):
    rows_per_tile = n // _NS
    mesh = plsc.VectorSubcoreMesh(core_axis_name="c", subcore_axis_name="s")

    nbuf = 8
    dpre = 4
    assert ch % nbuf >= dpre and ch > 2 * nbuf

    scratch = [
        pltpu.VMEM((ch, k), jnp.int32),
        pltpu.VMEM((ch, k), jnp.float32),
        pltpu.VMEM((nbuf, k, _L), jnp.float32),
        pltpu.VMEM((rows_per_tile, _L), jnp.float32),
        pltpu.VMEM_SHARED((n, _L), jnp.float32),
        [pltpu.SemaphoreType.DMA] * nbuf,
        [pltpu.SemaphoreType.DMA] * nbuf,
    ]
    if with_table:
        scratch.insert(0, pltpu.VMEM((ch, k), jnp.int32))

    def body(*refs):
        if with_table:
            (tab_hbm, row_hbm, col_hbm, ew_hbm, out_hbm,
             row_v, col_v, ew_v, msg_r, buf_v, acc, gsems, ssems) = refs
        else:
            (col_hbm, ew_hbm, out_hbm,
             col_v, ew_v, msg_r, buf_v, acc, gsems, ssems) = refs
        cid = lax.axis_index("c")
        sid = lax.axis_index("s")
        wid = sid * _NC + cid
        base = sid * rows_per_tile

        @pl.loop(0, rows_per_tile)
        def _zero(i):
            buf_v[i] = jnp.zeros((_L,), jnp.float32)

        pltpu.sync_copy(buf_v, acc.at[pl.ds(base, rows_per_tile)])

        if with_table:
            pltpu.sync_copy(row_hbm.at[wid], row_v)
        pltpu.sync_copy(col_hbm.at[wid], col_v)
        pltpu.sync_copy(ew_hbm.at[wid], ew_v)
        plsc.subcore_barrier()

        def gather(j, u):
            if with_table:
                pltpu.async_copy(tab_hbm.at[row_v.at[j]], msg_r.at[u], gsems[u])

        def gather_wait(j, u):
            if with_table:
                pltpu.make_async_copy(
                    tab_hbm.at[row_v.at[j]], msg_r.at[u], gsems[u]).wait()

        def scale(j, u):
            buf = msg_r.at[u]
            for g in range(k // _L):
                ewv = ew_v[j, pl.ds(g * _L, _L)]
                for t in range(_L):
                    i = g * _L + t
                    if with_table:
                        buf[i] = buf[i] * ewv[t]
                    else:
                        buf[i] = jnp.broadcast_to(ewv[t], (_L,))

        def scatter(j, u):
            pltpu.async_copy(msg_r.at[u], acc.at[col_v.at[j]], ssems[u], add=True)

        def scatter_wait(j, u):
            pltpu.make_async_copy(msg_r.at[u], acc.at[col_v.at[j]], ssems[u]).wait()

        main_ch = (ch // nbuf) * nbuf

        for j in range(dpre):
            gather(j, j)

        @pl.loop(0, ch // nbuf)
        def _round(r):
            jr = r * nbuf
            for u in range(nbuf):
                j = jr + u
                uf = (u + dpre) % nbuf
                if u < dpre:
                    @pl.when(r > 0)
                    def _drain():
                        scatter_wait(j - dpre, uf)
                else:
                    scatter_wait(j - dpre, uf)
                gather(j + dpre, uf)
                gather_wait(j, u)
                scale(j, u)
                scatter(j, u)

        for j in range(main_ch, ch):
            u = j % nbuf
            uf = (j + dpre) % nbuf
            if j + dpre < ch:
                scatter_wait(j - dpre, uf)
                gather(j + dpre, uf)
            gather_wait(j, u)
            scale(j, u)
            scatter(j, u)

        for j in range(ch - nbuf, ch):
            scatter_wait(j, j % nbuf)

        plsc.subcore_barrier()
        pltpu.sync_copy(acc.at[pl.ds(base, rows_per_tile)], buf_v)
        pltpu.sync_copy(buf_v, out_hbm.at[cid, pl.ds(base, rows_per_tile)])

    return pl.kernel(
        body,
        out_type=jax.ShapeDtypeStruct((_NC, n, _L), jnp.float32),
        mesh=mesh,
        scratch_types=scratch,
        compiler_params=pltpu.CompilerParams(use_tc_tiling_on_sc=False),
    )


def _tc1_body(x_ref, w1_ref, degp_ref, h1p_ref, dinv_ref):
    deg = degp_ref[0] + degp_ref[1] + 1.0
    dinv = jnp.where(deg > 0, lax.rsqrt(deg), 0.0)
    h1 = jnp.dot(x_ref[...], w1_ref[...], preferred_element_type=jnp.float32)
    h1p_ref[...] = dinv * h1
    dinv_ref[...] = dinv


def _tc2_body(s1p_ref, dinv_ref, h1p_ref, b1_ref, w2_ref, h2p_ref):
    dinv = dinv_ref[...]
    pre = dinv * (s1p_ref[0] + s1p_ref[1] + h1p_ref[...]) + b1_ref[...]
    out1 = jnp.maximum(pre, 0.0)
    h2 = jnp.dot(out1, w2_ref[...], preferred_element_type=jnp.float32)
    h2p_ref[...] = dinv * h2


def _tc3_body(s2p_ref, dinv_ref, h2p_ref, b2_ref, y_ref):
    pre = dinv_ref[...] * (s2p_ref[0] + s2p_ref[1] + h2p_ref[...]) + b2_ref[...]
    m = jnp.max(pre, axis=1, keepdims=True)
    shifted = pre - m
    lse = jnp.log(jnp.sum(jnp.exp(shifted), axis=1, keepdims=True))
    y_ref[...] = shifted - lse


def _tc_call(body, out_shapes):
    return pl.pallas_call(body, out_shape=out_shapes)


@functools.partial(jax.jit, static_argnames=())
def kernel(x, edge_index, edge_weight, W1, b1, W2, b2):
    n = x.shape[0]
    e = edge_weight.shape[0]
    k = 80
    assert e % (_NW * k) == 0 and n % _NS == 0
    ch = e // (_NW * k)

    row3 = edge_index[0].reshape(_NW, ch, k)
    col3 = edge_index[1].reshape(_NW, ch, k)
    ew3 = edge_weight.reshape(_NW, ch, k)
    b1r = b1.reshape(1, _L)
    b2r = b2.reshape(1, _L)

    deg_sc = _edge_scatter_sc(n, ch, k, with_table=False)
    msg_sc = _edge_scatter_sc(n, ch, k, with_table=True)
    f16 = jax.ShapeDtypeStruct((n, _L), jnp.float32)

    degp = deg_sc(col3, ew3)
    h1p, dinv = _tc_call(_tc1_body, (f16, f16))(x, W1, degp)
    s1p = msg_sc(h1p, row3, col3, ew3)
    h2p = _tc_call(_tc2_body, f16)(s1p, dinv, h1p, b1r, W2)
    s2p = msg_sc(h2p, row3, col3, ew3)
    return _tc_call(_tc3_body, f16)(s2p, dinv, h2p, b2r)
